# Initial kernel scaffold; baseline (speedup 1.0000x reference)
import jax
import jax.numpy as jnp
from jax.experimental import pallas as pl


def kernel(node_attr, edge_index, edge_attr, W1, b1, W2, b2):
    raise NotImplementedError("write your pallas kernel here")


# speedup vs baseline: 3.7534x; 3.7534x over previous
import functools

import jax
import jax.numpy as jnp
from jax import lax
from jax.experimental import pallas as pl
from jax.experimental.pallas import tpu as pltpu
from jax.experimental.pallas import tpu_sc as plsc

N_NODES = 10000
N_EDGES = 320000
D = 128

CHUNK = 128
NUM_CHUNKS = N_EDGES // CHUNK
NC = 2
NS = 16
NW = NC * NS
PER_W = 80
PAD_CHUNKS = NW * PER_W
ROWS_PER_TILE = (N_NODES // NS) // 8 * 8
TAIL_ROWS = N_NODES - ROWS_PER_TILE * NS


def _sc_scatter(edge_attr, rec2d, zeros_nd):
  mesh = plsc.VectorSubcoreMesh(core_axis_name="c", subcore_axis_name="s")

  @functools.partial(
      pl.kernel,
      mesh=mesh,
      out_type=jax.ShapeDtypeStruct((NC, N_NODES, D), jnp.float32),
      scratch_types=[
          pltpu.VMEM((PER_W, CHUNK), jnp.int32),
          pltpu.VMEM((CHUNK, D), jnp.float32),
          pltpu.VMEM_SHARED((N_NODES, D), jnp.float32),
      ],
  )
  def k(edge_hbm, rec_hbm, zero_hbm, out_hbm, rec_v, ebuf, acc_sh):
    cid = lax.axis_index("c")
    sid = lax.axis_index("s")
    wid = sid * NC + cid

    r0 = sid * ROWS_PER_TILE
    pltpu.sync_copy(zero_hbm.at[pl.ds(r0, ROWS_PER_TILE)],
                    acc_sh.at[pl.ds(r0, ROWS_PER_TILE)])

    @pl.when(sid == NS - 1)
    def _():
      pltpu.sync_copy(zero_hbm.at[pl.ds(ROWS_PER_TILE * NS, TAIL_ROWS)],
                      acc_sh.at[pl.ds(ROWS_PER_TILE * NS, TAIL_ROWS)])

    plsc.subcore_barrier()

    base = wid * PER_W
    cnt = jnp.clip(NUM_CHUNKS - base, 0, PER_W)
    pltpu.sync_copy(rec_hbm.at[pl.ds(base, PER_W)], rec_v)

    def body(j, carry):
      chunk = base + j
      pltpu.sync_copy(edge_hbm.at[pl.ds(chunk * CHUNK, CHUNK)], ebuf)
      pltpu.sync_copy(ebuf, acc_sh.at[rec_v.at[j]], add=True)
      return carry

    lax.fori_loop(0, cnt, body, 0)

    plsc.subcore_barrier()

    pltpu.sync_copy(acc_sh.at[pl.ds(r0, ROWS_PER_TILE)],
                    out_hbm.at[cid, pl.ds(r0, ROWS_PER_TILE)])

    @pl.when(sid == NS - 1)
    def _():
      pltpu.sync_copy(acc_sh.at[pl.ds(ROWS_PER_TILE * NS, TAIL_ROWS)],
                      out_hbm.at[cid, pl.ds(ROWS_PER_TILE * NS, TAIL_ROWS)])

  return k(edge_attr, rec2d, zeros_nd)


def _tc_mlp(node_attr, partials, w1a, w1b, b1, w2, b2):
  def body(node_ref, p_ref, w1a_ref, w1b_ref, b1_ref, w2_ref, b2_ref, out_ref):
    agg = p_ref[0] + p_ref[1]
    mean = jnp.sum(agg, axis=0, keepdims=True) * (1.0 / N_NODES)
    z = agg - mean
    h = jnp.dot(node_ref[...], w1a_ref[...],
                preferred_element_type=jnp.float32)
    h += jnp.dot(z, w1b_ref[...], preferred_element_type=jnp.float32)
    h = jnp.maximum(h + b1_ref[...], 0.0)
    out_ref[...] = jnp.dot(h, w2_ref[...],
                           preferred_element_type=jnp.float32) + b2_ref[...]

  return pl.pallas_call(
      body,
      out_shape=jax.ShapeDtypeStruct((N_NODES, D), jnp.float32),
  )(node_attr, partials, w1a, w1b, b1, w2, b2)


def kernel(node_attr, edge_index, edge_attr, W1, b1, W2, b2):
  receivers = edge_index[1]
  rec2d = jnp.zeros((PAD_CHUNKS, CHUNK), jnp.int32)
  rec2d = rec2d.at[:NUM_CHUNKS].set(receivers.reshape(NUM_CHUNKS, CHUNK))
  zeros_nd = jnp.zeros((N_NODES, D), jnp.float32)

  partials = _sc_scatter(edge_attr, rec2d, zeros_nd)

  w1a = W1[:D]
  w1b = W1[D:]
  x = _tc_mlp(node_attr, partials, w1a, w1b,
              b1.reshape(1, D), W2, b2.reshape(1, D))
  return (x, edge_index, edge_attr)


# speedup vs baseline: 4.7669x; 1.2700x over previous
import functools

import jax
import jax.numpy as jnp
from jax import lax
from jax.experimental import pallas as pl
from jax.experimental.pallas import tpu as pltpu
from jax.experimental.pallas import tpu_sc as plsc

N_NODES = 10000
N_EDGES = 320000
D = 128

CHUNK = 128
NUM_CHUNKS = N_EDGES // CHUNK
NC = 2
NS = 16
NW = NC * NS
PER_W = 80
PAD_CHUNKS = NW * PER_W
STEP = CHUNK
ROWS_PER_TILE = (N_NODES // NS) // 8 * 8
TAIL_ROWS = N_NODES - ROWS_PER_TILE * NS


def _sc_scatter(edge_attr, rec2d, zeros_nd):
  mesh = plsc.VectorSubcoreMesh(core_axis_name="c", subcore_axis_name="s")

  @functools.partial(
      pl.kernel,
      mesh=mesh,
      out_type=jax.ShapeDtypeStruct((NC, N_NODES, D), jnp.float32),
      scratch_types=[
          pltpu.VMEM((PER_W, CHUNK), jnp.int32),
          pltpu.VMEM((STEP, D), jnp.float32),
          pltpu.VMEM((STEP, D), jnp.float32),
          pltpu.VMEM_SHARED((N_NODES, D), jnp.float32),
          pltpu.SemaphoreType.DMA,
          pltpu.SemaphoreType.DMA,
      ],
  )
  def k(edge_hbm, rec_hbm, zero_hbm, out_hbm, rec_v, ebuf0, ebuf1, acc_sh,
        sem0, sem1):
    cid = lax.axis_index("c")
    sid = lax.axis_index("s")
    wid = sid * NC + cid

    r0 = sid * ROWS_PER_TILE
    pltpu.sync_copy(zero_hbm.at[pl.ds(r0, ROWS_PER_TILE)],
                    acc_sh.at[pl.ds(r0, ROWS_PER_TILE)])

    @pl.when(sid == NS - 1)
    def _():
      pltpu.sync_copy(zero_hbm.at[pl.ds(ROWS_PER_TILE * NS, TAIL_ROWS)],
                      acc_sh.at[pl.ds(ROWS_PER_TILE * NS, TAIL_ROWS)])

    plsc.subcore_barrier()

    base = wid * PER_W
    cnt = jnp.clip(NUM_CHUNKS - base, 0, PER_W)
    steps = cnt
    pairs = steps // 2
    base_e = base * CHUNK
    pltpu.sync_copy(rec_hbm.at[pl.ds(base, PER_W)], rec_v)

    def fetch(s, buf, sem):
      pltpu.make_async_copy(
          edge_hbm.at[pl.ds(base_e + s * STEP, STEP)], buf, sem).start()

    def drain(s, buf, sem):
      pltpu.make_async_copy(
          edge_hbm.at[pl.ds(0, STEP)], buf, sem).wait()
      pltpu.sync_copy(buf, acc_sh.at[rec_v.at[s]], add=True)

      @pl.when(s + 2 < steps)
      def _():
        fetch(s + 2, buf, sem)

    fetch(0, ebuf0, sem0)
    fetch(1, ebuf1, sem1)

    def body(p, carry):
      drain(2 * p, ebuf0, sem0)
      drain(2 * p + 1, ebuf1, sem1)
      return carry

    lax.fori_loop(0, pairs, body, 0)

    plsc.subcore_barrier()

    pltpu.sync_copy(acc_sh.at[pl.ds(r0, ROWS_PER_TILE)],
                    out_hbm.at[cid, pl.ds(r0, ROWS_PER_TILE)])

    @pl.when(sid == NS - 1)
    def _():
      pltpu.sync_copy(acc_sh.at[pl.ds(ROWS_PER_TILE * NS, TAIL_ROWS)],
                      out_hbm.at[cid, pl.ds(ROWS_PER_TILE * NS, TAIL_ROWS)])

  return k(edge_attr, rec2d, zeros_nd)


def _tc_mlp(node_attr, partials, w1a, w1b, b1, w2, b2):
  def body(node_ref, p_ref, w1a_ref, w1b_ref, b1_ref, w2_ref, b2_ref, out_ref):
    agg = p_ref[0] + p_ref[1]
    mean = jnp.sum(agg, axis=0, keepdims=True) * (1.0 / N_NODES)
    z = agg - mean
    h = jnp.dot(node_ref[...], w1a_ref[...],
                preferred_element_type=jnp.float32)
    h += jnp.dot(z, w1b_ref[...], preferred_element_type=jnp.float32)
    h = jnp.maximum(h + b1_ref[...], 0.0)
    out_ref[...] = jnp.dot(h, w2_ref[...],
                           preferred_element_type=jnp.float32) + b2_ref[...]

  return pl.pallas_call(
      body,
      out_shape=jax.ShapeDtypeStruct((N_NODES, D), jnp.float32),
  )(node_attr, partials, w1a, w1b, b1, w2, b2)


def kernel(node_attr, edge_index, edge_attr, W1, b1, W2, b2):
  receivers = edge_index[1]
  rec2d = jnp.zeros((PAD_CHUNKS, CHUNK), jnp.int32)
  rec2d = rec2d.at[:NUM_CHUNKS].set(receivers.reshape(NUM_CHUNKS, CHUNK))
  zeros_nd = jnp.zeros((N_NODES, D), jnp.float32)

  partials = _sc_scatter(edge_attr, rec2d, zeros_nd)

  w1a = W1[:D]
  w1b = W1[D:]
  x = _tc_mlp(node_attr, partials, w1a, w1b,
              b1.reshape(1, D), W2, b2.reshape(1, D))
  return (x, edge_index, edge_attr)
